# Initial kernel scaffold; baseline (speedup 1.0000x reference)
import jax
import jax.numpy as jnp
from jax.experimental import pallas as pl


def kernel(bert_outputs, token_idxs, token_masks, token_nums, W1, b1, W2, b2, transitions, start_trans, end_trans):
    raise NotImplementedError("write your pallas kernel here")


# speedup vs baseline: 4.2799x; 4.2799x over previous
import functools

import jax
import jax.numpy as jnp
from jax import lax
from jax.experimental import pallas as pl
from jax.experimental.pallas import tpu as pltpu
from jax.experimental.pallas import tpu_sc as plsc

NEG = -1e30


def _make_sc_gather(n_rows, D, W):
    NW = 32
    RPW = n_rows // NW
    CH = 16
    NCH = RPW // CH
    GR = CH * W

    mesh = plsc.VectorSubcoreMesh(core_axis_name="c", subcore_axis_name="s")

    @functools.partial(
        pl.kernel,
        mesh=mesh,
        out_type=jax.ShapeDtypeStruct((n_rows, D), jnp.float32),
        scratch_types=[
            pltpu.VMEM((GR,), jnp.int32),
            pltpu.VMEM((GR, D), jnp.float32),
            pltpu.VMEM((GR, 16), jnp.float32),
            pltpu.VMEM((CH, D), jnp.float32),
            pltpu.SemaphoreType.DMA,
        ],
    )
    def sc_gather(bert_hbm, idx_hbm, msk_hbm, out_hbm, idx_v, rows_v, m_v,
                  out_v, sem):
        wid = lax.axis_index("s") * 2 + lax.axis_index("c")

        def chunk_body(ci, carry):
            base_out = wid * RPW + ci * CH
            base_g = base_out * W
            pltpu.sync_copy(idx_hbm.at[pl.ds(base_g, GR)], idx_v)
            cp = pltpu.async_copy(bert_hbm.at[idx_v], rows_v, sem)
            pltpu.sync_copy(msk_hbm.at[pl.ds(base_g, GR)], m_v)
            cp.wait()
            for j in range(CH):
                m0 = m_v[2 * j]
                m1 = m_v[2 * j + 1]

                def col_body(c, _, j=j, m0=m0, m1=m1):
                    s = pl.ds(c * 16, 16)
                    out_v[j, s] = rows_v[2 * j, s] * m0 + rows_v[2 * j + 1, s] * m1
                    return 0

                lax.fori_loop(0, D // 16, col_body, 0)
            pltpu.sync_copy(out_v, out_hbm.at[pl.ds(base_out, CH)])
            return carry

        lax.fori_loop(0, NCH, chunk_body, 0)

    return sc_gather


def _make_tc_mlp_crf(B, T, D, H, KP, RT):
    NT = (T * B) // RT

    def body(x_ref, w1_ref, b1_ref, w2_ref, b2_ref, tr_ref, st_ref, en_ref,
             len_ref, out_ref, sc_scr):
        i = pl.program_id(0)
        x = x_ref[...]
        h = jnp.maximum(
            jnp.dot(x, w1_ref[...], preferred_element_type=jnp.float32)
            + b1_ref[...], 0.0)
        s = jnp.dot(h, w2_ref[...], preferred_element_type=jnp.float32) \
            + b2_ref[...]
        sc_scr[pl.ds(i * RT, RT), :] = s

        @pl.when(i == NT - 1)
        def _():
            tr = tr_ref[...]
            ct = jnp.max(tr, axis=0, keepdims=True)
            expT = jnp.exp(tr - ct)
            lens = len_ref[...]
            alpha0 = sc_scr[pl.ds(0, B), :] + st_ref[...]

            def step(t, alpha):
                emit = sc_scr[pl.ds(t * B, B), :]
                m = jnp.max(alpha, axis=1, keepdims=True)
                p = jnp.exp(alpha - m)
                q = jnp.dot(p, expT, preferred_element_type=jnp.float32)
                na = emit + m + ct + jnp.log(q)
                return jnp.where(t < lens, na, alpha)

            alpha = lax.fori_loop(1, T, step, alpha0)
            alpha = alpha + en_ref[...]
            m2 = jnp.max(alpha, axis=1, keepdims=True)
            z = m2 + jnp.log(
                jnp.sum(jnp.exp(alpha - m2), axis=1, keepdims=True))
            out_ref[...] = jnp.broadcast_to(z, (B, KP))

    return body, NT


def kernel(bert_outputs, token_idxs, token_masks, token_nums, W1, b1, W2, b2,
           transitions, start_trans, end_trans):
    B, L, D = bert_outputs.shape
    TW = token_idxs.shape[1]
    W = 2
    T = TW // W
    H = W1.shape[1]
    K = W2.shape[1]
    KP = 128

    bert_flat = bert_outputs.reshape(B * L, D)
    idx = token_idxs.astype(jnp.int32) + (
        jnp.arange(B, dtype=jnp.int32) * L)[:, None]
    idx_tb = idx.reshape(B, T, W).transpose(1, 0, 2).reshape(T * B * W)
    msk_tb = token_masks.reshape(B, T, W).transpose(1, 0, 2).reshape(
        T * B * W)
    msk_b = jnp.broadcast_to(msk_tb[:, None], (T * B * W, 16))

    W2p = jnp.zeros((H, KP), jnp.float32).at[:, :K].set(W2)
    b2p = jnp.zeros((1, KP), jnp.float32).at[0, :K].set(b2)
    trp = jnp.full((KP, KP), NEG, jnp.float32).at[:K, :K].set(transitions)
    stp = jnp.full((1, KP), NEG, jnp.float32).at[0, :K].set(start_trans)
    enp = jnp.full((1, KP), NEG, jnp.float32).at[0, :K].set(end_trans)
    lens = jnp.maximum(token_nums, 1).astype(jnp.int32)
    lens2d = jnp.broadcast_to(lens[:, None], (B, KP))
    b1_2d = b1.reshape(1, H)

    sc_gather = _make_sc_gather(T * B, D, W)
    reprs = sc_gather(bert_flat, idx_tb, msk_b)

    RT = 512
    body, NT = _make_tc_mlp_crf(B, T, D, H, KP, RT)
    out = pl.pallas_call(
        body,
        grid=(NT,),
        in_specs=[
            pl.BlockSpec((RT, D), lambda i: (i, 0)),
            pl.BlockSpec((D, H), lambda i: (0, 0)),
            pl.BlockSpec((1, H), lambda i: (0, 0)),
            pl.BlockSpec((H, KP), lambda i: (0, 0)),
            pl.BlockSpec((1, KP), lambda i: (0, 0)),
            pl.BlockSpec((KP, KP), lambda i: (0, 0)),
            pl.BlockSpec((1, KP), lambda i: (0, 0)),
            pl.BlockSpec((1, KP), lambda i: (0, 0)),
            pl.BlockSpec((B, KP), lambda i: (0, 0)),
        ],
        out_specs=pl.BlockSpec((B, KP), lambda i: (0, 0)),
        out_shape=jax.ShapeDtypeStruct((B, KP), jnp.float32),
        scratch_shapes=[pltpu.VMEM((T * B, KP), jnp.float32)],
        compiler_params=pltpu.CompilerParams(
            dimension_semantics=("arbitrary",)),
    )(reprs, W1, b1_2d, W2p, b2p, trp, stp, enp, lens2d)
    return out[:, 0]
